# baseline (speedup 1.0000x reference)
import functools

import jax
import jax.numpy as jnp
from jax import lax
from jax.experimental import pallas as pl
from jax.experimental.pallas import tpu as pltpu
from jax.experimental.pallas import tpu_sc as plsc

_N = 10000
_D = 512
_H = 2
_C = 512
_HC = _H * _C

_NC = 2
_NS = 16
_NW = _NC * _NS

_B = 16
_N_ACC = 10240
_RPW = _N_ACC // _NS

_BN = 2000


def _matmul_stage(x, W_l, W_r, Wc):
    n = x.shape[0]
    grid = n // _BN

    def mm_kernel(x_ref, wl_ref, wr_ref, wc_ref, xl_ref, xr_ref, y_ref):
        xb = x_ref[...].astype(jnp.bfloat16)
        xl = jnp.dot(xb, wl_ref[...].astype(jnp.bfloat16),
                     preferred_element_type=jnp.float32)
        xr = jnp.dot(xb, wr_ref[...].astype(jnp.bfloat16),
                     preferred_element_type=jnp.float32)
        xl_ref[...] = xl.astype(jnp.bfloat16)
        xr_ref[...] = xr.astype(jnp.bfloat16)
        y0 = jnp.dot(xl[:, :_C], wc_ref[...], preferred_element_type=jnp.float32)
        y1 = jnp.dot(xl[:, _C:], wc_ref[...], preferred_element_type=jnp.float32)
        y_ref[...] = jnp.concatenate(
            [y0, y1, jnp.zeros((_BN, 12), jnp.float32)], axis=1)

    return pl.pallas_call(
        mm_kernel,
        grid=(grid,),
        in_specs=[
            pl.BlockSpec((_BN, _D), lambda i: (i, 0)),
            pl.BlockSpec((_D, _HC), lambda i: (0, 0)),
            pl.BlockSpec((_D, _HC), lambda i: (0, 0)),
            pl.BlockSpec((_D, 2), lambda i: (0, 0)),
        ],
        out_specs=[
            pl.BlockSpec((_BN, _HC), lambda i: (i, 0)),
            pl.BlockSpec((_BN, _HC), lambda i: (i, 0)),
            pl.BlockSpec((_BN, 16), lambda i: (i, 0)),
        ],
        out_shape=[
            jax.ShapeDtypeStruct((n, _HC), jnp.bfloat16),
            jax.ShapeDtypeStruct((n, _HC), jnp.bfloat16),
            jax.ShapeDtypeStruct((n, 16), jnp.float32),
        ],
    )(x, W_l, W_r, Wc)


def _make_sc_edge_kernel(e_pad):
    epw = e_pad // _NW
    nblk = epw // _B
    mesh = plsc.VectorSubcoreMesh(
        core_axis_name="c", subcore_axis_name="s",
        num_cores=_NC, num_subcores=_NS)

    @functools.partial(
        pl.kernel,
        mesh=mesh,
        compiler_params=pltpu.CompilerParams(
            needs_layout_passes=False, use_tc_tiling_on_sc=False),
        out_type=jax.ShapeDtypeStruct((_NC, _N_ACC, 8), jnp.float32),
        scratch_types=[
            pltpu.VMEM((epw,), jnp.int32),
            pltpu.VMEM((epw,), jnp.int32),
            pltpu.VMEM((epw,), jnp.int32),
            pltpu.VMEM((_HC,), jnp.bfloat16),
            pltpu.VMEM((_B, _HC), jnp.bfloat16),
            pltpu.VMEM((_B, _HC), jnp.bfloat16),
            pltpu.VMEM((_B, _HC), jnp.bfloat16),
            pltpu.VMEM((_B, _HC), jnp.bfloat16),
            pltpu.VMEM((_B, _HC), jnp.bfloat16),
            pltpu.VMEM((_B, _HC), jnp.bfloat16),
            pltpu.VMEM((_B, 16), jnp.float32),
            pltpu.VMEM((_B, 16), jnp.float32),
            pltpu.VMEM((_B, 16), jnp.float32),
            pltpu.VMEM((_B, 8), jnp.float32),
            pltpu.VMEM((_RPW, 8), jnp.float32),
            pltpu.VMEM_SHARED((_N_ACC, 8), jnp.float32),
            pltpu.SemaphoreType.DMA,
            pltpu.SemaphoreType.DMA,
            pltpu.SemaphoreType.DMA,
        ],
    )
    def sc_edge(xl_hbm, xr_hbm, y_hbm, src_hbm, dstg_hbm, dsts_hbm, att_hbm,
                zero_hbm, acc_hbm, srcv, dgv, dsv, attv, xlb0, xlb1, xlb2,
                xrb0, xrb1, xrb2, yb0, yb1, yb2, contrib, obuf, acc_sh,
                sem0, sem1, sem2):
        xlb_s = [xlb0, xlb1, xlb2]
        xrb_s = [xrb0, xrb1, xrb2]
        yb_s = [yb0, yb1, yb2]
        sem_s = [sem0, sem1, sem2]
        cid = lax.axis_index("c")
        sid = lax.axis_index("s")
        wid = sid * _NC + cid
        base_e = wid * epw

        pltpu.sync_copy(src_hbm.at[pl.ds(base_e, epw)], srcv)
        pltpu.sync_copy(dstg_hbm.at[pl.ds(base_e, epw)], dgv)
        pltpu.sync_copy(dsts_hbm.at[pl.ds(base_e, epw)], dsv)
        pltpu.sync_copy(att_hbm, attv)
        pltpu.sync_copy(zero_hbm, acc_sh.at[pl.ds(sid * _RPW, _RPW)])
        plsc.subcore_barrier()

        iota16 = lax.iota(jnp.int32, 16)
        zero16 = jnp.zeros((16,), jnp.float32)
        plsc.store_scatter(contrib, [iota16, jnp.full((16,), 6, jnp.int32)], zero16)
        plsc.store_scatter(contrib, [iota16, jnp.full((16,), 7, jnp.int32)], zero16)

        def issue(b, slot):
            off = b * _B
            src16 = srcv[pl.ds(off, _B)]
            dg16 = dgv[pl.ds(off, _B)]
            pltpu.async_copy(xl_hbm.at[src16], xlb_s[slot], sem_s[slot])
            pltpu.async_copy(xr_hbm.at[dg16], xrb_s[slot], sem_s[slot])
            pltpu.async_copy(y_hbm.at[src16], yb_s[slot], sem_s[slot])

        def wait_slot(slot):
            idx0 = srcv[pl.ds(0, _B)]
            pltpu.make_async_copy(
                xl_hbm.at[idx0], xlb_s[slot], sem_s[slot]).wait()
            pltpu.make_async_copy(
                xr_hbm.at[idx0], xrb_s[slot], sem_s[slot]).wait()
            pltpu.make_async_copy(
                y_hbm.at[idx0], yb_s[slot], sem_s[slot]).wait()

        def compute_block(b, slot):
            xlb = xlb_s[slot]
            xrb = xrb_s[slot]
            yb = yb_s[slot]
            off = b * _B
            ds16 = dsv[pl.ds(off, _B)]

            def edge_body(ed, evs):
                neg_slope = jnp.bfloat16(0.2)
                zero32b = jnp.zeros((32,), jnp.bfloat16)

                def term(o):
                    m = xlb[ed, pl.ds(o, 32)] + xrb[ed, pl.ds(o, 32)]
                    return attv[pl.ds(o, 32)] * jnp.maximum(m, neg_slope * m)

                def chunk(j, accs):
                    a0, a1 = accs
                    o = j * 32
                    return (a0 + term(o), a1 + term(_C + o))

                quarter = _C // 128
                accs = []
                for q in range(4):
                    a0, a1 = zero32b, zero32b
                    for j in range(q * quarter, (q + 1) * quarter):
                        a0, a1 = chunk(j, (a0, a1))
                    accs.append((a0, a1))

                def up(acc):
                    ua, ub = plsc.unpack(
                        acc, format=plsc.PackFormat.INTERLEAVED,
                        preferred_element_type=jnp.float32)
                    return ua + ub

                e0 = jnp.sum((up(accs[0][0]) + up(accs[1][0]))
                             + (up(accs[2][0]) + up(accs[3][0])))
                e1 = jnp.sum((up(accs[0][1]) + up(accs[1][1]))
                             + (up(accs[2][1]) + up(accs[3][1])))
                ev0, ev1 = evs
                lane = iota16 == ed
                return (jnp.where(lane, e0, ev0),
                        jnp.where(lane, e1, ev1))

            ev0, ev1 = lax.fori_loop(0, _B, edge_body, (zero16, zero16),
                                     unroll=2)
            col = lambda k: jnp.full((16,), k, jnp.int32)
            ee0 = jnp.exp(ev0)
            ee1 = jnp.exp(ev1)
            y00 = plsc.load_gather(yb, [iota16, col(0)])
            y01 = plsc.load_gather(yb, [iota16, col(1)])
            y10 = plsc.load_gather(yb, [iota16, col(2)])
            y11 = plsc.load_gather(yb, [iota16, col(3)])
            plsc.store_scatter(contrib, [iota16, col(0)], ee0)
            plsc.store_scatter(contrib, [iota16, col(1)], ee1)
            plsc.store_scatter(contrib, [iota16, col(2)], ee0 * y00)
            plsc.store_scatter(contrib, [iota16, col(3)], ee0 * y01)
            plsc.store_scatter(contrib, [iota16, col(4)], ee1 * y10)
            plsc.store_scatter(contrib, [iota16, col(5)], ee1 * y11)
            pltpu.sync_copy(contrib, acc_sh.at[ds16], add=True)

        issue(0, 0)
        issue(1, 1)

        def triple_body(i, carry):
            b0 = i * 3
            issue(b0 + 2, 2)
            wait_slot(0)
            compute_block(b0, 0)

            @pl.when(b0 + 3 < nblk)
            def _():
                issue(b0 + 3, 0)

            wait_slot(1)
            compute_block(b0 + 1, 1)

            @pl.when(b0 + 4 < nblk)
            def _():
                issue(b0 + 4, 1)

            wait_slot(2)
            compute_block(b0 + 2, 2)
            return carry

        lax.fori_loop(0, nblk // 3, triple_body, 0)
        plsc.subcore_barrier()

        pltpu.sync_copy(acc_sh.at[pl.ds(sid * _RPW, _RPW)], obuf)
        pltpu.sync_copy(obuf, acc_hbm.at[cid, pl.ds(sid * _RPW, _RPW)])

    return sc_edge


def _combine_stage(acc, Wc, bias, bc):
    grid = _N_ACC // _BN
    bn = 1024
    grid = _N_ACC // bn

    def fin_kernel(acc_ref, wc_ref, bias_ref, bc_ref, out_ref):
        a = acc_ref[0] + acc_ref[1]
        d0 = a[:, 0:1]
        d1 = a[:, 1:2]
        n0 = a[:, 2:4]
        n1 = a[:, 4:6]
        part = 0.5 * (n0 / (d0 + 1e-16) + n1 / (d1 + 1e-16))
        const = jnp.dot(bias_ref[...], wc_ref[...],
                        preferred_element_type=jnp.float32) + bc_ref[...]
        out_ref[...] = part + const

    return pl.pallas_call(
        fin_kernel,
        grid=(grid,),
        in_specs=[
            pl.BlockSpec((_NC, bn, 8), lambda i: (0, i, 0)),
            pl.BlockSpec((_D, 2), lambda i: (0, 0)),
            pl.BlockSpec((1, _D), lambda i: (0, 0)),
            pl.BlockSpec((1, 2), lambda i: (0, 0)),
        ],
        out_specs=pl.BlockSpec((bn, 2), lambda i: (i, 0)),
        out_shape=jax.ShapeDtypeStruct((_N_ACC, 2), jnp.float32),
    )(acc, Wc, bias.reshape(1, _D), bc.reshape(1, 2))


def kernel(x, edge_index, W_l, W_r, att, bias, ln_gamma, ln_beta, Wc, bc):
    n = x.shape[0]
    e = edge_index.shape[1]
    e_full = e + n
    blk3 = _NW * _B * 3
    e_pad = ((e_full + blk3 - 1) // blk3) * blk3
    npad = e_pad - e_full

    loop = jnp.arange(n, dtype=jnp.int32)
    src = jnp.concatenate(
        [edge_index[0], loop, jnp.zeros((npad,), jnp.int32)])
    dstg = jnp.concatenate(
        [edge_index[1], loop, jnp.zeros((npad,), jnp.int32)])
    dsts = jnp.concatenate(
        [edge_index[1], loop, jnp.full((npad,), n, jnp.int32)])

    xl, xr, y = _matmul_stage(x, W_l, W_r, Wc)
    sc_edge = _make_sc_edge_kernel(e_pad)
    acc = sc_edge(xl, xr, y, src, dstg, dsts,
                  att.reshape(-1).astype(jnp.bfloat16),
                  jnp.zeros((_RPW, 8), jnp.float32))
    logits_pad = _combine_stage(acc, Wc, bias, bc)
    return logits_pad[:n]


# speedup vs baseline: 2.3732x; 2.3732x over previous
import functools

import jax
import jax.numpy as jnp
from jax import lax
from jax.experimental import pallas as pl
from jax.experimental.pallas import tpu as pltpu
from jax.experimental.pallas import tpu_sc as plsc

_N = 10000
_D = 512
_H = 2
_C = 512
_HC = _H * _C

_NC = 2
_NS = 16
_NW = _NC * _NS

_B = 16
_N_ACC = 10240
_RPW = _N_ACC // _NS

_BN = 2000


def _matmul_stage(x, W_l, W_r, Wc):
    n = x.shape[0]
    grid = n // _BN

    def mm_kernel(x_ref, wl_ref, wr_ref, wc_ref, xl_ref, xr_ref, y_ref):
        xb = x_ref[...].astype(jnp.bfloat16)
        xl = jnp.dot(xb, wl_ref[...].astype(jnp.bfloat16),
                     preferred_element_type=jnp.float32)
        xr = jnp.dot(xb, wr_ref[...].astype(jnp.bfloat16),
                     preferred_element_type=jnp.float32)
        xl_ref[...] = xl.astype(jnp.bfloat16)
        xr_ref[...] = xr.astype(jnp.bfloat16)
        y0 = jnp.dot(xl[:, :_C], wc_ref[...], preferred_element_type=jnp.float32)
        y1 = jnp.dot(xl[:, _C:], wc_ref[...], preferred_element_type=jnp.float32)
        y_ref[...] = jnp.concatenate(
            [y0, y1, jnp.zeros((_BN, 12), jnp.float32)], axis=1)

    return pl.pallas_call(
        mm_kernel,
        grid=(grid,),
        in_specs=[
            pl.BlockSpec((_BN, _D), lambda i: (i, 0)),
            pl.BlockSpec((_D, _HC), lambda i: (0, 0)),
            pl.BlockSpec((_D, _HC), lambda i: (0, 0)),
            pl.BlockSpec((_D, 2), lambda i: (0, 0)),
        ],
        out_specs=[
            pl.BlockSpec((_BN, _HC), lambda i: (i, 0)),
            pl.BlockSpec((_BN, _HC), lambda i: (i, 0)),
            pl.BlockSpec((_BN, 16), lambda i: (i, 0)),
        ],
        out_shape=[
            jax.ShapeDtypeStruct((n, _HC), jnp.bfloat16),
            jax.ShapeDtypeStruct((n, _HC), jnp.bfloat16),
            jax.ShapeDtypeStruct((n, 16), jnp.float32),
        ],
    )(x, W_l, W_r, Wc)


def _make_sc_edge_kernel(e_pad):
    epw = e_pad // _NW
    nblk = epw // _B
    mesh = plsc.VectorSubcoreMesh(
        core_axis_name="c", subcore_axis_name="s",
        num_cores=_NC, num_subcores=_NS)

    @functools.partial(
        pl.kernel,
        mesh=mesh,
        compiler_params=pltpu.CompilerParams(
            needs_layout_passes=False, use_tc_tiling_on_sc=False),
        out_type=jax.ShapeDtypeStruct((_NC, _N_ACC, 8), jnp.float32),
        scratch_types=[
            pltpu.VMEM((epw,), jnp.int32),
            pltpu.VMEM((epw,), jnp.int32),
            pltpu.VMEM((epw,), jnp.int32),
            pltpu.VMEM((_HC,), jnp.bfloat16),
            pltpu.VMEM((_B, _HC), jnp.bfloat16),
            pltpu.VMEM((_B, _HC), jnp.bfloat16),
            pltpu.VMEM((_B, _HC), jnp.bfloat16),
            pltpu.VMEM((_B, _HC), jnp.bfloat16),
            pltpu.VMEM((_B, _HC), jnp.bfloat16),
            pltpu.VMEM((_B, _HC), jnp.bfloat16),
            pltpu.VMEM((_B, 16), jnp.float32),
            pltpu.VMEM((_B, 16), jnp.float32),
            pltpu.VMEM((_B, 16), jnp.float32),
            pltpu.VMEM((_B, 8), jnp.float32),
            pltpu.VMEM((_RPW, 8), jnp.float32),
            pltpu.VMEM_SHARED((_N_ACC, 8), jnp.float32),
            pltpu.SemaphoreType.DMA,
            pltpu.SemaphoreType.DMA,
            pltpu.SemaphoreType.DMA,
        ],
    )
    def sc_edge(xl_hbm, xr_hbm, y_hbm, src_hbm, dstg_hbm, dsts_hbm, att_hbm,
                zero_hbm, acc_hbm, srcv, dgv, dsv, attv, xlb0, xlb1, xlb2,
                xrb0, xrb1, xrb2, yb0, yb1, yb2, contrib, obuf, acc_sh,
                sem0, sem1, sem2):
        xlb_s = [xlb0, xlb1, xlb2]
        xrb_s = [xrb0, xrb1, xrb2]
        yb_s = [yb0, yb1, yb2]
        sem_s = [sem0, sem1, sem2]
        cid = lax.axis_index("c")
        sid = lax.axis_index("s")
        wid = sid * _NC + cid
        base_e = wid * epw

        pltpu.sync_copy(src_hbm.at[pl.ds(base_e, epw)], srcv)
        pltpu.sync_copy(dstg_hbm.at[pl.ds(base_e, epw)], dgv)
        pltpu.sync_copy(dsts_hbm.at[pl.ds(base_e, epw)], dsv)
        pltpu.sync_copy(att_hbm, attv)
        pltpu.sync_copy(zero_hbm, acc_sh.at[pl.ds(sid * _RPW, _RPW)])
        plsc.subcore_barrier()

        iota16 = lax.iota(jnp.int32, 16)
        zero16 = jnp.zeros((16,), jnp.float32)
        plsc.store_scatter(contrib, [iota16, jnp.full((16,), 6, jnp.int32)], zero16)
        plsc.store_scatter(contrib, [iota16, jnp.full((16,), 7, jnp.int32)], zero16)

        def issue(b, slot):
            off = b * _B
            src16 = srcv[pl.ds(off, _B)]
            dg16 = dgv[pl.ds(off, _B)]
            pltpu.async_copy(xl_hbm.at[src16], xlb_s[slot], sem_s[slot])
            pltpu.async_copy(xr_hbm.at[dg16], xrb_s[slot], sem_s[slot])
            pltpu.async_copy(y_hbm.at[src16], yb_s[slot], sem_s[slot])

        def wait_slot(slot):
            idx0 = srcv[pl.ds(0, _B)]
            pltpu.make_async_copy(
                xl_hbm.at[idx0], xlb_s[slot], sem_s[slot]).wait()
            pltpu.make_async_copy(
                xr_hbm.at[idx0], xrb_s[slot], sem_s[slot]).wait()
            pltpu.make_async_copy(
                y_hbm.at[idx0], yb_s[slot], sem_s[slot]).wait()

        def compute_block(b, slot):
            xlb = xlb_s[slot]
            xrb = xrb_s[slot]
            yb = yb_s[slot]
            off = b * _B
            ds16 = dsv[pl.ds(off, _B)]

            def edge_body(ed, evs):
                neg_slope = jnp.bfloat16(0.2)
                zero32b = jnp.zeros((32,), jnp.bfloat16)

                def term(o):
                    m = xlb[ed, pl.ds(o, 32)] + xrb[ed, pl.ds(o, 32)]
                    return attv[pl.ds(o, 32)] * jnp.maximum(m, neg_slope * m)

                def chunk(j, accs):
                    a0, a1 = accs
                    o = j * 32
                    return (a0 + term(o), a1 + term(_C + o))

                quarter = _C // 128
                accs = []
                for q in range(4):
                    a0, a1 = zero32b, zero32b
                    for j in range(q * quarter, (q + 1) * quarter):
                        a0, a1 = chunk(j, (a0, a1))
                    accs.append((a0, a1))

                def up(acc):
                    ua, ub = plsc.unpack(
                        acc, format=plsc.PackFormat.INTERLEAVED,
                        preferred_element_type=jnp.float32)
                    return ua + ub

                e0 = jnp.sum((up(accs[0][0]) + up(accs[1][0]))
                             + (up(accs[2][0]) + up(accs[3][0])))
                e1 = jnp.sum((up(accs[0][1]) + up(accs[1][1]))
                             + (up(accs[2][1]) + up(accs[3][1])))
                ev0, ev1 = evs
                lane = iota16 == ed
                return (jnp.where(lane, e0, ev0),
                        jnp.where(lane, e1, ev1))

            ev0, ev1 = lax.fori_loop(0, _B, edge_body, (zero16, zero16))
            col = lambda k: jnp.full((16,), k, jnp.int32)
            ee0 = jnp.exp(ev0)
            ee1 = jnp.exp(ev1)
            y00 = plsc.load_gather(yb, [iota16, col(0)])
            y01 = plsc.load_gather(yb, [iota16, col(1)])
            y10 = plsc.load_gather(yb, [iota16, col(2)])
            y11 = plsc.load_gather(yb, [iota16, col(3)])
            plsc.store_scatter(contrib, [iota16, col(0)], ee0)
            plsc.store_scatter(contrib, [iota16, col(1)], ee1)
            plsc.store_scatter(contrib, [iota16, col(2)], ee0 * y00)
            plsc.store_scatter(contrib, [iota16, col(3)], ee0 * y01)
            plsc.store_scatter(contrib, [iota16, col(4)], ee1 * y10)
            plsc.store_scatter(contrib, [iota16, col(5)], ee1 * y11)
            pltpu.sync_copy(contrib, acc_sh.at[ds16], add=True)

        issue(0, 0)
        issue(1, 1)

        def triple_body(i, carry):
            b0 = i * 3
            issue(b0 + 2, 2)
            wait_slot(0)
            compute_block(b0, 0)

            @pl.when(b0 + 3 < nblk)
            def _():
                issue(b0 + 3, 0)

            wait_slot(1)
            compute_block(b0 + 1, 1)

            @pl.when(b0 + 4 < nblk)
            def _():
                issue(b0 + 4, 1)

            wait_slot(2)
            compute_block(b0 + 2, 2)
            return carry

        lax.fori_loop(0, nblk // 3, triple_body, 0)
        plsc.subcore_barrier()

        pltpu.sync_copy(acc_sh.at[pl.ds(sid * _RPW, _RPW)], obuf)
        pltpu.sync_copy(obuf, acc_hbm.at[cid, pl.ds(sid * _RPW, _RPW)])

    return sc_edge


def _combine_stage(acc, Wc, bias, bc):
    grid = _N_ACC // _BN
    bn = 1024
    grid = _N_ACC // bn

    def fin_kernel(acc_ref, wc_ref, bias_ref, bc_ref, out_ref):
        a = acc_ref[0] + acc_ref[1]
        d0 = a[:, 0:1]
        d1 = a[:, 1:2]
        n0 = a[:, 2:4]
        n1 = a[:, 4:6]
        part = 0.5 * (n0 / (d0 + 1e-16) + n1 / (d1 + 1e-16))
        const = jnp.dot(bias_ref[...], wc_ref[...],
                        preferred_element_type=jnp.float32) + bc_ref[...]
        out_ref[...] = part + const

    return pl.pallas_call(
        fin_kernel,
        grid=(grid,),
        in_specs=[
            pl.BlockSpec((_NC, bn, 8), lambda i: (0, i, 0)),
            pl.BlockSpec((_D, 2), lambda i: (0, 0)),
            pl.BlockSpec((1, _D), lambda i: (0, 0)),
            pl.BlockSpec((1, 2), lambda i: (0, 0)),
        ],
        out_specs=pl.BlockSpec((bn, 2), lambda i: (i, 0)),
        out_shape=jax.ShapeDtypeStruct((_N_ACC, 2), jnp.float32),
    )(acc, Wc, bias.reshape(1, _D), bc.reshape(1, 2))


def kernel(x, edge_index, W_l, W_r, att, bias, ln_gamma, ln_beta, Wc, bc):
    n = x.shape[0]
    e = edge_index.shape[1]
    e_full = e + n
    blk3 = _NW * _B * 3
    e_pad = ((e_full + blk3 - 1) // blk3) * blk3
    npad = e_pad - e_full

    loop = jnp.arange(n, dtype=jnp.int32)
    src = jnp.concatenate(
        [edge_index[0], loop, jnp.zeros((npad,), jnp.int32)])
    dstg = jnp.concatenate(
        [edge_index[1], loop, jnp.zeros((npad,), jnp.int32)])
    dsts = jnp.concatenate(
        [edge_index[1], loop, jnp.full((npad,), n, jnp.int32)])

    xl, xr, y = _matmul_stage(x, W_l, W_r, Wc)
    sc_edge = _make_sc_edge_kernel(e_pad)
    acc = sc_edge(xl, xr, y, src, dstg, dsts,
                  att.reshape(-1).astype(jnp.bfloat16),
                  jnp.zeros((_RPW, 8), jnp.float32))
    logits_pad = _combine_stage(acc, Wc, bias, bc)
    return logits_pad[:n]
